# baseline (speedup 1.0000x reference)
import functools

import jax
import jax.numpy as jnp
from jax import lax
from jax.experimental import pallas as pl
from jax.experimental.pallas import tpu as pltpu
from jax.experimental.pallas import tpu_sc as plsc

NC = 2
NS = 16
NW = NC * NS

D = 32
V = 1000000
NB = 16384
SEQ = 50
B_TOTAL = NB * SEQ
B_PER_W = B_TOTAL // NW
NB_PER_W = NB // NW
BGRP = 32
HALF = BGRP // 2
CHUNK_H = HALF * SEQ
N_GROUPS = NB_PER_W // BGRP

VT_FULL = V // 128
V_TAIL = V - VT_FULL * 128
KT = VT_FULL // NW


def _make_transpose():
  mesh = plsc.VectorSubcoreMesh(core_axis_name="c", subcore_axis_name="s")

  @functools.partial(
      pl.kernel,
      mesh=mesh,
      out_type=jax.ShapeDtypeStruct((V, D), jnp.float32),
      scratch_types=[
          pltpu.VMEM((D // 8, 8, 128), jnp.float32),
          pltpu.VMEM((D // 8, 8, 128), jnp.float32),
          pltpu.VMEM((128, D), jnp.float32),
          pltpu.VMEM((128, D), jnp.float32),
          pltpu.SemaphoreType.DMA,
          pltpu.SemaphoreType.DMA,
          pltpu.SemaphoreType.DMA,
          pltpu.SemaphoreType.DMA,
      ],
      compiler_params=pltpu.CompilerParams(
          use_tc_tiling_on_sc=True, needs_layout_passes=False
      ),
  )
  def transpose_kernel(wt_hbm, tail_hbm, out_hbm, a0, a1, b0, b1,
                       sa0, sa1, sb0, sb1):
    aa = (a0, a1)
    bb = (b0, b1)
    sa = (sa0, sa1)
    sb = (sb0, sb1)

    wid = lax.axis_index("s") * NC + lax.axis_index("c")

    lane = lax.iota(jnp.int32, 16)
    ft_lo = lax.shift_right_logical(lane, 3)
    ft_hi = ft_lo + 2
    fi = lax.bitwise_and(lane, 7)

    def tile_of(k):
      return k * NW + wid

    def load(k, s):
      t = tile_of(k)
      v0 = t * 128
      for ft in range(D // 8):
        pltpu.make_async_copy(
            wt_hbm.at[pl.ds(ft * 8, 8), pl.ds(v0, 128)], aa[s].at[ft], sa[s]
        ).start()

    def load_wait(k, s):
      t = tile_of(k)
      v0 = t * 128
      for ft in range(D // 8):
        pltpu.make_async_copy(
            wt_hbm.at[pl.ds(ft * 8, 8), pl.ds(v0, 128)], aa[s].at[ft], sa[s]
        ).wait()

    def store(k, s):
      v0 = tile_of(k) * 128
      return pltpu.make_async_copy(bb[s], out_hbm.at[pl.ds(v0, 128)], sb[s])

    def transpose(s, nv):
      a = aa[s]
      b = bb[s]

      def vrow(j, carry):
        v0 = j * 8
        vs = []
        for dv in range(8):
          cv = jnp.full((16,), 0, jnp.int32) + (v0 + dv)
          vs.append(plsc.load_gather(a, [ft_lo, fi, cv]))
          vs.append(plsc.load_gather(a, [ft_hi, fi, cv]))
        for dv in range(8):
          b[v0 + dv, pl.ds(0, 16)] = vs[2 * dv]
          b[v0 + dv, pl.ds(16, 16)] = vs[2 * dv + 1]
        return carry

      lax.fori_loop(0, nv // 8, vrow, 0)

    load(0, 0)
    load(1, 1)
    load_wait(0, 0)
    transpose(0, 128)
    store(0, 0).start()
    load(2, 0)
    load_wait(1, 1)
    transpose(1, 128)
    store(1, 1).start()
    load(3, 1)

    def body(j, carry):
      for s in range(2):
        k = j * 2 + s
        load_wait(k, s)
        store(k - 2, s).wait()
        transpose(s, 128)
        store(k, s).start()
        load(k + 2, s)
      return carry

    lax.fori_loop(1, KT // 2 - 1, body, 0)

    jl = KT // 2 - 1
    for s in range(2):
      k = jl * 2 + s
      load_wait(k, s)
      store(k - 2, s).wait()
      transpose(s, 128)
      store(k, s).start()
    for s in range(2):
      store(jl * 2 + s, s).wait()

    @pl.when(wid < VT_FULL - KT * NW)
    def _():
      load(KT, 0)
      load_wait(KT, 0)
      transpose(0, 128)
      store(KT, 0).start()
      store(KT, 0).wait()

    @pl.when(wid == NW - 1)
    def _():
      v0 = VT_FULL * 128
      pltpu.sync_copy(tail_hbm, bb[1].at[pl.ds(0, V_TAIL)])
      pltpu.sync_copy(bb[1].at[pl.ds(0, V_TAIL)], out_hbm.at[pl.ds(v0, V_TAIL)])

  return transpose_kernel


def _make_gather():
  mesh = plsc.VectorSubcoreMesh(core_axis_name="c", subcore_axis_name="s")

  @functools.partial(
      pl.kernel,
      mesh=mesh,
      out_type=jax.ShapeDtypeStruct((SEQ, D // 8, NB // 128, 8, 128),
                                    jnp.float32),
      scratch_types=[
          pltpu.VMEM((B_PER_W,), jnp.int32),
          pltpu.VMEM((CHUNK_H, D), jnp.float32),
          pltpu.VMEM((CHUNK_H, D), jnp.float32),
          pltpu.VMEM((SEQ, D // 8, 8, BGRP), jnp.float32),
          pltpu.SemaphoreType.DMA,
          pltpu.SemaphoreType.DMA,
          pltpu.SemaphoreType.DMA,
      ],
      compiler_params=pltpu.CompilerParams(
          use_tc_tiling_on_sc=False, needs_layout_passes=False
      ),
  )
  def gather_kernel(idx_hbm, table_hbm, out_hbm, idx_all, rows0, rows1, trs,
                    sem_g0, sem_g1, sem_o):
    rows = (rows0, rows1)
    sem_g = (sem_g0, sem_g1)

    wid = lax.axis_index("s") * NC + lax.axis_index("c")
    base_w = wid * B_PER_W
    base_g = wid * N_GROUPS

    pltpu.sync_copy(idx_hbm.at[pl.ds(base_w, B_PER_W)], idx_all)

    lane = lax.iota(jnp.int32, 16)
    row_base = lane * SEQ

    def gather(g, h):
      off = pl.multiple_of(g * (2 * CHUNK_H) + h * CHUNK_H, 8)
      src = table_hbm.at[idx_all.at[pl.ds(off, CHUNK_H)]]
      return pltpu.make_async_copy(src, rows[h], sem_g[h])

    def store(g):
      gg = base_g + g
      bt = gg // 4
      bi0 = (gg % 4) * BGRP
      dst = out_hbm.at[:, :, bt, :, pl.ds(bi0, BGRP)]
      return pltpu.make_async_copy(trs, dst, sem_o)

    def transpose(h):
      r = rows[h]

      def srow(s, carry):
        ridx = row_base + s
        vs = [
            plsc.load_gather(r, [ridx, jnp.full((16,), f, jnp.int32)])
            for f in range(D)
        ]
        for f in range(D):
          trs[s, f // 8, f % 8, pl.ds(h * HALF, HALF)] = vs[f]
        return carry

      lax.fori_loop(0, SEQ, srow, 0)

    gather(0, 0).start()
    gather(0, 1).start()
    gather(0, 0).wait()
    transpose(0)
    gather(1, 0).start()
    gather(0, 1).wait()
    transpose(1)
    store(0).start()
    gather(1, 1).start()

    def body(g, carry):
      gather(g, 0).wait()
      store(g - 1).wait()
      transpose(0)
      gather(g + 1, 0).start()
      gather(g, 1).wait()
      transpose(1)
      store(g).start()
      gather(g + 1, 1).start()
      return carry

    lax.fori_loop(1, N_GROUPS - 1, body, 0)

    gl = N_GROUPS - 1
    gather(gl, 0).wait()
    store(gl - 1).wait()
    transpose(0)
    gather(gl, 1).wait()
    transpose(1)
    store(gl).start()
    store(gl).wait()

  return gather_kernel


_transpose = _make_transpose()
_gather = _make_gather()


@jax.jit
def kernel(weights, indices):
  idx_flat = indices.reshape(-1).astype(jnp.int32)
  tail = lax.slice(weights, (VT_FULL * 128, 0), (V, D))
  w_rm = _transpose(weights.T, tail)
  out6 = _gather(idx_flat, w_rm)
  t = lax.transpose(out6, (2, 4, 0, 1, 3))
  return t.reshape(NB, SEQ, D)


# speedup vs baseline: 1.0022x; 1.0022x over previous
import functools

import jax
import jax.numpy as jnp
from jax import lax
from jax.experimental import pallas as pl
from jax.experimental.pallas import tpu as pltpu
from jax.experimental.pallas import tpu_sc as plsc

NC = 2
NS = 16
NW = NC * NS

D = 32
V = 1000000
NB = 16384
SEQ = 50
B_TOTAL = NB * SEQ
B_PER_W = B_TOTAL // NW
NB_PER_W = NB // NW
BGRP = 32
HALF = BGRP // 2
CHUNK_H = HALF * SEQ
N_GROUPS = NB_PER_W // BGRP

VT_FULL = V // 128
V_TAIL = V - VT_FULL * 128
KT = VT_FULL // NW


def _make_transpose():
  mesh = plsc.VectorSubcoreMesh(core_axis_name="c", subcore_axis_name="s")

  @functools.partial(
      pl.kernel,
      mesh=mesh,
      out_type=jax.ShapeDtypeStruct((V, D), jnp.float32),
      scratch_types=[
          pltpu.VMEM((D, 128), jnp.float32),
          pltpu.VMEM((D, 128), jnp.float32),
          pltpu.VMEM((128, D), jnp.float32),
          pltpu.VMEM((128, D), jnp.float32),
          pltpu.SemaphoreType.DMA,
          pltpu.SemaphoreType.DMA,
          pltpu.SemaphoreType.DMA,
          pltpu.SemaphoreType.DMA,
      ],
      compiler_params=pltpu.CompilerParams(
          use_tc_tiling_on_sc=True, needs_layout_passes=False
      ),
  )
  def transpose_kernel(wt_hbm, tail_hbm, out_hbm, a0, a1, b0, b1,
                       sa0, sa1, sb0, sb1):
    aa = (a0, a1)
    bb = (b0, b1)
    sa = (sa0, sa1)
    sb = (sb0, sb1)

    wid = lax.axis_index("s") * NC + lax.axis_index("c")

    lane = lax.iota(jnp.int32, 16)
    ft_lo = lax.shift_right_logical(lane, 3)
    ft_hi = ft_lo + 2
    fi = lax.bitwise_and(lane, 7)

    def tile_of(k):
      return k * NW + wid

    def load(k, s):
      v0 = tile_of(k) * 128
      pltpu.make_async_copy(
          wt_hbm.at[:, pl.ds(v0, 128)], aa[s], sa[s]
      ).start()

    def load_wait(k, s):
      v0 = tile_of(k) * 128
      pltpu.make_async_copy(
          wt_hbm.at[:, pl.ds(v0, 128)], aa[s], sa[s]
      ).wait()

    def store(k, s):
      v0 = tile_of(k) * 128
      return pltpu.make_async_copy(bb[s], out_hbm.at[pl.ds(v0, 128)], sb[s])

    def transpose(s, nv):
      a = aa[s]
      b = bb[s]

      def vrow(j, carry):
        v0 = j * 8
        vs = []
        for dv in range(8):
          cv = jnp.full((16,), 0, jnp.int32) + (v0 + dv)
          vs.append(plsc.load_gather(a, [lane, cv]))
          vs.append(plsc.load_gather(a, [lane + 16, cv]))
        for dv in range(8):
          b[v0 + dv, pl.ds(0, 16)] = vs[2 * dv]
          b[v0 + dv, pl.ds(16, 16)] = vs[2 * dv + 1]
        return carry

      lax.fori_loop(0, nv // 8, vrow, 0)

    load(0, 0)
    load(1, 1)
    load_wait(0, 0)
    transpose(0, 128)
    store(0, 0).start()
    load(2, 0)
    load_wait(1, 1)
    transpose(1, 128)
    store(1, 1).start()
    load(3, 1)

    def body(j, carry):
      for s in range(2):
        k = j * 2 + s
        load_wait(k, s)
        store(k - 2, s).wait()
        transpose(s, 128)
        store(k, s).start()
        load(k + 2, s)
      return carry

    lax.fori_loop(1, KT // 2 - 1, body, 0)

    jl = KT // 2 - 1
    for s in range(2):
      k = jl * 2 + s
      load_wait(k, s)
      store(k - 2, s).wait()
      transpose(s, 128)
      store(k, s).start()
    for s in range(2):
      store(jl * 2 + s, s).wait()

    @pl.when(wid < VT_FULL - KT * NW)
    def _():
      load(KT, 0)
      load_wait(KT, 0)
      transpose(0, 128)
      store(KT, 0).start()
      store(KT, 0).wait()

    @pl.when(wid == NW - 1)
    def _():
      v0 = VT_FULL * 128
      pltpu.sync_copy(tail_hbm, bb[1].at[pl.ds(0, V_TAIL)])
      pltpu.sync_copy(bb[1].at[pl.ds(0, V_TAIL)], out_hbm.at[pl.ds(v0, V_TAIL)])

  return transpose_kernel


def _make_gather():
  mesh = plsc.VectorSubcoreMesh(core_axis_name="c", subcore_axis_name="s")

  @functools.partial(
      pl.kernel,
      mesh=mesh,
      out_type=jax.ShapeDtypeStruct((SEQ, D // 8, NB // 128, 8, 128),
                                    jnp.float32),
      scratch_types=[
          pltpu.VMEM((B_PER_W,), jnp.int32),
          pltpu.VMEM((CHUNK_H, D), jnp.float32),
          pltpu.VMEM((CHUNK_H, D), jnp.float32),
          pltpu.VMEM((SEQ, D // 8, 8, BGRP), jnp.float32),
          pltpu.SemaphoreType.DMA,
          pltpu.SemaphoreType.DMA,
          pltpu.SemaphoreType.DMA,
      ],
      compiler_params=pltpu.CompilerParams(
          use_tc_tiling_on_sc=False, needs_layout_passes=False
      ),
  )
  def gather_kernel(idx_hbm, table_hbm, out_hbm, idx_all, rows0, rows1, trs,
                    sem_g0, sem_g1, sem_o):
    rows = (rows0, rows1)
    sem_g = (sem_g0, sem_g1)

    wid = lax.axis_index("s") * NC + lax.axis_index("c")
    base_w = wid * B_PER_W
    base_g = wid * N_GROUPS

    pltpu.sync_copy(idx_hbm.at[pl.ds(base_w, B_PER_W)], idx_all)

    lane = lax.iota(jnp.int32, 16)
    row_base = lane * SEQ

    def gather(g, h):
      off = pl.multiple_of(g * (2 * CHUNK_H) + h * CHUNK_H, 8)
      src = table_hbm.at[idx_all.at[pl.ds(off, CHUNK_H)]]
      return pltpu.make_async_copy(src, rows[h], sem_g[h])

    def store(g):
      gg = base_g + g
      bt = gg // 4
      bi0 = (gg % 4) * BGRP
      dst = out_hbm.at[:, :, bt, :, pl.ds(bi0, BGRP)]
      return pltpu.make_async_copy(trs, dst, sem_o)

    def transpose(h):
      r = rows[h]

      def srow(s, carry):
        ridx = row_base + s
        vs = [
            plsc.load_gather(r, [ridx, jnp.full((16,), f, jnp.int32)])
            for f in range(D)
        ]
        for f in range(D):
          trs[s, f // 8, f % 8, pl.ds(h * HALF, HALF)] = vs[f]
        return carry

      lax.fori_loop(0, SEQ, srow, 0)

    gather(0, 0).start()
    gather(0, 1).start()
    gather(0, 0).wait()
    transpose(0)
    gather(1, 0).start()
    gather(0, 1).wait()
    transpose(1)
    store(0).start()
    gather(1, 1).start()

    def body(g, carry):
      gather(g, 0).wait()
      store(g - 1).wait()
      transpose(0)
      gather(g + 1, 0).start()
      gather(g, 1).wait()
      transpose(1)
      store(g).start()
      gather(g + 1, 1).start()
      return carry

    lax.fori_loop(1, N_GROUPS - 1, body, 0)

    gl = N_GROUPS - 1
    gather(gl, 0).wait()
    store(gl - 1).wait()
    transpose(0)
    gather(gl, 1).wait()
    transpose(1)
    store(gl).start()
    store(gl).wait()

  return gather_kernel


_transpose = _make_transpose()
_gather = _make_gather()


@jax.jit
def kernel(weights, indices):
  idx_flat = indices.reshape(-1).astype(jnp.int32)
  tail = lax.slice(weights, (VT_FULL * 128, 0), (V, D))
  w_rm = _transpose(weights.T, tail)
  out6 = _gather(idx_flat, w_rm)
  t = lax.transpose(out6, (2, 4, 0, 1, 3))
  return t.reshape(NB, SEQ, D)


# speedup vs baseline: 1.3882x; 1.3852x over previous
import functools

import jax
import jax.numpy as jnp
from jax import lax
from jax.experimental import pallas as pl
from jax.experimental.pallas import tpu as pltpu
from jax.experimental.pallas import tpu_sc as plsc

NC = 2
NS = 16
NW = NC * NS

D = 32
NB = 16384
SEQ = 50
B_TOTAL = NB * SEQ
B_PER_W = B_TOTAL // NW
NB_PER_W = NB // NW
BGRP = 32
HALF = BGRP // 2
CHUNK_H = HALF * SEQ
N_GROUPS = NB_PER_W // BGRP


def _make_kernel():
  mesh = plsc.VectorSubcoreMesh(core_axis_name="c", subcore_axis_name="s")

  @functools.partial(
      pl.kernel,
      mesh=mesh,
      out_type=jax.ShapeDtypeStruct((SEQ, D // 8, NB // 128, 8, 128),
                                    jnp.float32),
      scratch_types=[
          pltpu.VMEM((B_PER_W,), jnp.int32),
          pltpu.VMEM((CHUNK_H, D), jnp.float32),
          pltpu.VMEM((CHUNK_H, D), jnp.float32),
          pltpu.VMEM((SEQ, D // 8, 8, BGRP), jnp.float32),
          pltpu.SemaphoreType.DMA,
          pltpu.SemaphoreType.DMA,
          pltpu.SemaphoreType.DMA,
      ],
      compiler_params=pltpu.CompilerParams(
          use_tc_tiling_on_sc=False, needs_layout_passes=False
      ),
  )
  def gather_kernel(idx_hbm, table_hbm, out_hbm, idx_all, rows0, rows1, trs,
                    sem_g0, sem_g1, sem_o):
    rows = (rows0, rows1)
    sem_g = (sem_g0, sem_g1)

    wid = lax.axis_index("s") * NC + lax.axis_index("c")
    base_w = wid * B_PER_W
    base_g = wid * N_GROUPS

    pltpu.sync_copy(idx_hbm.at[pl.ds(base_w, B_PER_W)], idx_all)

    lane = lax.iota(jnp.int32, 16)
    row_base = lane * SEQ

    def gather(g, h):
      off = pl.multiple_of(g * (2 * CHUNK_H) + h * CHUNK_H, 8)
      src = table_hbm.at[idx_all.at[pl.ds(off, CHUNK_H)]]
      return pltpu.make_async_copy(src, rows[h], sem_g[h])

    def store(g):
      gg = base_g + g
      bt = gg // 4
      bi0 = (gg % 4) * BGRP
      dst = out_hbm.at[:, :, bt, :, pl.ds(bi0, BGRP)]
      return pltpu.make_async_copy(trs, dst, sem_o)

    def transpose(h):
      r = rows[h]

      def srow(s, carry):
        ridx = row_base + s
        vs = [
            plsc.load_gather(r, [ridx, jnp.full((16,), f, jnp.int32)])
            for f in range(D)
        ]
        for f in range(D):
          trs[s, f // 8, f % 8, pl.ds(h * HALF, HALF)] = vs[f]
        return carry

      lax.fori_loop(0, SEQ, srow, 0, unroll=2)

    gather(0, 0).start()
    gather(0, 1).start()
    gather(0, 0).wait()
    transpose(0)
    gather(1, 0).start()
    gather(0, 1).wait()
    transpose(1)
    store(0).start()
    gather(1, 1).start()

    def body(g, carry):
      gather(g, 0).wait()
      store(g - 1).wait()
      transpose(0)
      gather(g + 1, 0).start()
      gather(g, 1).wait()
      transpose(1)
      store(g).start()
      gather(g + 1, 1).start()
      return carry

    lax.fori_loop(1, N_GROUPS - 1, body, 0)

    gl = N_GROUPS - 1
    gather(gl, 0).wait()
    store(gl - 1).wait()
    transpose(0)
    gather(gl, 1).wait()
    transpose(1)
    store(gl).start()
    store(gl).wait()

  return gather_kernel


_gather = _make_kernel()


@jax.jit
def kernel(weights, indices):
  idx_flat = indices.reshape(-1).astype(jnp.int32)
  out6 = _gather(idx_flat, weights)
  t = lax.transpose(out6, (2, 4, 0, 1, 3))
  return t.reshape(NB, SEQ, D)


# speedup vs baseline: 1.3919x; 1.0027x over previous
import functools

import jax
import jax.numpy as jnp
from jax import lax
from jax.experimental import pallas as pl
from jax.experimental.pallas import tpu as pltpu
from jax.experimental.pallas import tpu_sc as plsc

NC = 2
NS = 16
NW = NC * NS

D = 32
NB = 16384
SEQ = 50
B_TOTAL = NB * SEQ
B_PER_W = B_TOTAL // NW
NB_PER_W = NB // NW
BGRP = 32
HALF = BGRP // 2
CHUNK_H = HALF * SEQ
N_GROUPS = NB_PER_W // BGRP


def _make_kernel():
  mesh = plsc.VectorSubcoreMesh(core_axis_name="c", subcore_axis_name="s")

  @functools.partial(
      pl.kernel,
      mesh=mesh,
      out_type=jax.ShapeDtypeStruct((SEQ, D // 8, NB // 128, 8, 128),
                                    jnp.float32),
      scratch_types=[
          pltpu.VMEM((B_PER_W,), jnp.int32),
          pltpu.VMEM((CHUNK_H, D), jnp.float32),
          pltpu.VMEM((CHUNK_H, D), jnp.float32),
          pltpu.VMEM((SEQ, D // 8, 8, BGRP), jnp.float32),
          pltpu.SemaphoreType.DMA,
          pltpu.SemaphoreType.DMA,
          pltpu.SemaphoreType.DMA,
      ],
      compiler_params=pltpu.CompilerParams(
          use_tc_tiling_on_sc=False, needs_layout_passes=False
      ),
  )
  def gather_kernel(idx_hbm, table_hbm, out_hbm, idx_all, rows0, rows1, trs,
                    sem_g0, sem_g1, sem_o):
    rows = (rows0, rows1)
    sem_g = (sem_g0, sem_g1)

    wid = lax.axis_index("s") * NC + lax.axis_index("c")
    base_w = wid * B_PER_W
    base_g = wid * N_GROUPS

    pltpu.sync_copy(idx_hbm.at[pl.ds(base_w, B_PER_W)], idx_all)

    lane = lax.iota(jnp.int32, 16)
    row_base = lane * SEQ

    def gather(g, h):
      off = pl.multiple_of(g * (2 * CHUNK_H) + h * CHUNK_H, 8)
      src = table_hbm.at[idx_all.at[pl.ds(off, CHUNK_H)]]
      return pltpu.make_async_copy(src, rows[h], sem_g[h])

    def store(g):
      gg = base_g + g
      bt = gg // 4
      bi0 = (gg % 4) * BGRP
      dst = out_hbm.at[:, :, bt, :, pl.ds(bi0, BGRP)]
      return pltpu.make_async_copy(trs, dst, sem_o)

    def transpose(h):
      r = rows[h]

      def srow(s, carry):
        ridx = row_base + s
        vs = [
            plsc.load_gather(r, [ridx, jnp.full((16,), f, jnp.int32)])
            for f in range(D)
        ]
        for f in range(D):
          trs[s, f // 8, f % 8, pl.ds(h * HALF, HALF)] = vs[f]
        return carry

      lax.fori_loop(0, SEQ, srow, 0)

    gather(0, 0).start()
    gather(0, 1).start()
    gather(0, 0).wait()
    transpose(0)
    gather(1, 0).start()
    gather(0, 1).wait()
    transpose(1)
    store(0).start()
    gather(1, 1).start()

    def body(g, carry):
      gather(g, 0).wait()
      store(g - 1).wait()
      transpose(0)
      gather(g + 1, 0).start()
      gather(g, 1).wait()
      transpose(1)
      store(g).start()
      gather(g + 1, 1).start()
      return carry

    lax.fori_loop(1, N_GROUPS - 1, body, 0)

    gl = N_GROUPS - 1
    gather(gl, 0).wait()
    store(gl - 1).wait()
    transpose(0)
    gather(gl, 1).wait()
    transpose(1)
    store(gl).start()
    store(gl).wait()

  return gather_kernel


_gather = _make_kernel()


@jax.jit
def kernel(weights, indices):
  idx_flat = indices.reshape(-1).astype(jnp.int32)
  out6 = _gather(idx_flat, weights)
  t = lax.transpose(out6, (2, 4, 0, 1, 3))
  return t.reshape(NB, SEQ, D)
